# Initial kernel scaffold; baseline (speedup 1.0000x reference)
import jax
import jax.numpy as jnp
from jax.experimental import pallas as pl


def kernel(x):
    raise NotImplementedError("write your pallas kernel here")


# speedup vs baseline: 2.8036x; 2.8036x over previous
import functools

import jax
import jax.numpy as jnp
import numpy as np
from jax import lax
from jax.experimental import pallas as pl
from jax.experimental.pallas import tpu as pltpu
from jax.experimental.pallas import tpu_sc as plsc

_T, _B, _D = 8192, 4, 1024
_DF = _B * _D
_N1 = int(_T * 0.7)
_N2 = _T - _N1
_C = 8
_NC, _NS = 2, 16
_NW = _NC * _NS


def _ceil_div(a, b):
    return -(-a // b)


_PERM = np.asarray(jax.random.permutation(jax.random.key(42), _T))
_IDX1 = np.sort(_PERM[:_N1]).astype(np.int32)
_IDX2 = np.sort(_PERM[_N1:]).astype(np.int32)

_WPW1 = _ceil_div(_ceil_div(_N1, _C), _NW)
_WPW1 += _WPW1 % 2
_WPW2 = _ceil_div(_ceil_div(_N2, _C), _NW)
_WPW2 += _WPW2 % 2


def _window_idx(idx, n, wpw):
    tot = _NW * wpw
    out = np.empty((tot, _C), np.int32)
    for w in range(tot):
        base = min(w * _C, n - _C)
        out[w] = idx[base:base + _C]
    return out.reshape(-1)


_R1 = _window_idx(_IDX1, _N1, _WPW1)
_R2 = _window_idx(_IDX2, _N2, _WPW2)


def _sc_body(x_ref, r1_ref, r2_ref, o1_ref, o2_ref,
             idx1_v, idx2_v, buf0, buf1, gsem0, gsem1, ssem0, ssem1):
    wid = lax.axis_index("s") * _NC + lax.axis_index("c")
    bufs = (buf0, buf1)
    gsems = (gsem0, gsem1)
    ssems = (ssem0, ssem1)

    pltpu.sync_copy(r1_ref.at[pl.ds(wid * (_WPW1 * _C), _WPW1 * _C)], idx1_v)
    pltpu.sync_copy(r2_ref.at[pl.ds(wid * (_WPW2 * _C), _WPW2 * _C)], idx2_v)

    def phase(idx_v, out_ref, wpw, nmin):
        base_w = wid * wpw

        def g_copy(r, b):
            return pltpu.make_async_copy(
                x_ref.at[idx_v.at[pl.ds(r * _C, _C)]], bufs[b], gsems[b])

        for b in range(2):
            g_copy(b, b).start()

        def body(i, carry):
            for b in range(2):
                r = i * 2 + b
                g_copy(r, b).wait()
                obase = jnp.minimum((base_w + r) * _C, nmin)
                s_copy = pltpu.make_async_copy(
                    bufs[b], out_ref.at[pl.ds(obase, _C)], ssems[b])
                s_copy.start()
                s_copy.wait()
                g_copy(jnp.minimum(r + 2, wpw - 1), b).start()
            return carry

        lax.fori_loop(0, wpw // 2, body, 0)
        for b in range(2):
            g_copy(wpw - 1, b).wait()

    phase(idx1_v, o1_ref, _WPW1, _N1 - _C)
    phase(idx2_v, o2_ref, _WPW2, _N2 - _C)


@functools.cache
def _sc_call():
    return functools.partial(
        pl.kernel,
        out_type=(
            jax.ShapeDtypeStruct((_N1, _B, _D), jnp.float32),
            jax.ShapeDtypeStruct((_N2, _B, _D), jnp.float32),
        ),
        mesh=plsc.VectorSubcoreMesh(core_axis_name="c", subcore_axis_name="s",
                                    num_cores=_NC, num_subcores=_NS),
        scratch_types=[
            pltpu.VMEM((_WPW1 * _C,), jnp.int32),
            pltpu.VMEM((_WPW2 * _C,), jnp.int32),
            pltpu.VMEM((_C, _B, _D), jnp.float32),
            pltpu.VMEM((_C, _B, _D), jnp.float32),
            pltpu.SemaphoreType.DMA,
            pltpu.SemaphoreType.DMA,
            pltpu.SemaphoreType.DMA,
            pltpu.SemaphoreType.DMA,
        ],
    )(_sc_body)


def kernel(x):
    return _sc_call()(x, jnp.asarray(_R1), jnp.asarray(_R2))
